# Initial kernel scaffold; baseline (speedup 1.0000x reference)
import jax
import jax.numpy as jnp
from jax.experimental import pallas as pl


def kernel(table, idx, targets):
    raise NotImplementedError("write your pallas kernel here")


# speedup vs baseline: 1.3784x; 1.3784x over previous
import functools

import jax
import jax.numpy as jnp
from jax import lax
from jax.experimental import pallas as pl
from jax.experimental.pallas import tpu as pltpu
from jax.experimental.pallas import tpu_sc as plsc

V = 8192
D = 8192
NTOK = 8192

NC, NS = 2, 16
NW = NC * NS
TPW = NTOK // NW
CH = 4
NCH = TPW // CH

_MESH = dict(core_axis_name="c", subcore_axis_name="s", num_cores=NC,
             num_subcores=NS)


def _gather_rows(table, idx3):
    mesh = plsc.VectorSubcoreMesh(**_MESH)

    @functools.partial(
        pl.kernel, mesh=mesh,
        out_type=jax.ShapeDtypeStruct((NTOK, D), jnp.float32),
        scratch_types=[
            pltpu.VMEM((NCH, CH), jnp.int32),
            pltpu.VMEM((CH, D), jnp.float32),
            pltpu.VMEM((CH, D), jnp.float32),
            pltpu.SemaphoreType.DMA,
            pltpu.SemaphoreType.DMA,
        ],
    )
    def k(table_hbm, idx_hbm, out_hbm, idx_v, buf0, buf1, sem0, sem1):
        wid = lax.axis_index("s") * NC + lax.axis_index("c")
        base = wid * TPW
        pltpu.sync_copy(idx_hbm.at[wid], idx_v)

        def start(c, buf, sem):
            pltpu.async_copy(table_hbm.at[idx_v.at[c]], buf, sem)

        def wait(buf, sem):
            pltpu.make_async_copy(table_hbm.at[idx_v.at[0]], buf, sem).wait()

        def put(c, buf):
            pltpu.sync_copy(buf, out_hbm.at[pl.ds(base + c * CH, CH)])

        start(0, buf0, sem0)

        def pair(p, carry):
            c0 = 2 * p
            start(c0 + 1, buf1, sem1)
            wait(buf0, sem0)
            put(c0, buf0)

            @pl.when(p < NCH // 2 - 1)
            def _():
                start(c0 + 2, buf0, sem0)

            wait(buf1, sem1)
            put(c0 + 1, buf1)
            return carry

        lax.fori_loop(0, NCH // 2, pair, 0)

    return k(table, idx3)


def _row_lse(table):
    R = 256

    def body(x_ref, o_ref):
        x = x_ref[...]
        m = jnp.max(x, axis=1, keepdims=True)
        s = jnp.sum(jnp.exp(x - m), axis=1, keepdims=True)
        o_ref[...] = (m[:, 0] + jnp.log(s[:, 0]))[None, None, :]

    return pl.pallas_call(
        body,
        grid=(V // R,),
        in_specs=[pl.BlockSpec((R, D), lambda i: (i, 0))],
        out_specs=pl.BlockSpec((1, 1, R), lambda i: (i, 0, 0)),
        out_shape=jax.ShapeDtypeStruct((V // R, 1, R), jnp.float32),
    )(table)


def _loss_partials(table_flat, idx2, tgt2, lse):
    mesh = plsc.VectorSubcoreMesh(**_MESH)

    @functools.partial(
        pl.kernel, mesh=mesh,
        out_type=jax.ShapeDtypeStruct((NW, 16), jnp.float32),
        scratch_types=[
            pltpu.VMEM((TPW,), jnp.int32),
            pltpu.VMEM((TPW,), jnp.int32),
            pltpu.VMEM((TPW,), jnp.int32),
            pltpu.VMEM((TPW,), jnp.float32),
            pltpu.VMEM((TPW,), jnp.float32),
            pltpu.VMEM((16,), jnp.float32),
            pltpu.SemaphoreType.DMA,
        ],
    )
    def k(tf_hbm, idx_hbm, tgt_hbm, lse_hbm, out_hbm,
          idx_v, tgt_v, flat_v, picked_v, lsetok_v, acc_v, sem):
        wid = lax.axis_index("s") * NC + lax.axis_index("c")
        pltpu.sync_copy(idx_hbm.at[wid], idx_v)
        pltpu.sync_copy(tgt_hbm.at[wid], tgt_v)
        for kk in range(TPW // 16):
            sl = pl.ds(kk * 16, 16)
            flat_v[sl] = idx_v[sl] * D + tgt_v[sl]
        for h in range(TPW // 128):
            hs = pl.ds(h * 128, 128)
            pltpu.async_copy(tf_hbm.at[flat_v.at[hs]], picked_v.at[hs],
                             sem).wait()
            pltpu.async_copy(lse_hbm.at[idx_v.at[hs]], lsetok_v.at[hs],
                             sem).wait()
        acc = jnp.zeros((16,), jnp.float32)
        for kk in range(TPW // 16):
            sl = pl.ds(kk * 16, 16)
            acc = acc + (lsetok_v[sl] - picked_v[sl])
        acc_v[...] = acc
        pltpu.sync_copy(acc_v, out_hbm.at[wid])

    return k(table_flat, idx2, tgt2, lse)


def _finalize(parts):
    def body(p_ref, o_ref):
        o_ref[...] = (jnp.sum(p_ref[...]) * (1.0 / NTOK))[None, None]

    return pl.pallas_call(
        body,
        out_shape=jax.ShapeDtypeStruct((1, 1), jnp.float32),
    )(parts)


def kernel(table, idx, targets):
    idx_i = idx.reshape(-1).astype(jnp.int32)
    tgt_i = targets.reshape(-1).astype(jnp.int32)
    idx3 = idx_i.reshape(NW, NCH, CH)
    idx2 = idx_i.reshape(NW, TPW)
    tgt2 = tgt_i.reshape(NW, TPW)

    logits2 = _gather_rows(table, idx3)
    lse = _row_lse(table).reshape(-1)
    parts = _loss_partials(table.reshape(-1), idx2, tgt2, lse)
    loss = _finalize(parts).reshape(())
    return (logits2, loss)
